# baseline (speedup 1.0000x reference)
import functools

import jax
import jax.numpy as jnp
from jax import lax
from jax.experimental import pallas as pl
from jax.experimental.pallas import tpu as pltpu
from jax.experimental.pallas import tpu_sc as plsc

NN = 10000
EE = 160000
DD = 256
AA = 128
NPAD = 10240
EPAD = 163840
KB = 64
ROWS_B = EPAD // KB
BT = ROWS_B // 16
BH = ROWS_B // 32
IC = 16
SLAB = NPAD // 16
KX = 40
XROWS = NPAD // KX
XBT = XROWS // 32
BN = 400
GRID = NN // BN

_sc_mesh = plsc.VectorSubcoreMesh(core_axis_name="c", subcore_axis_name="s")


@functools.partial(
    pl.kernel,
    mesh=_sc_mesh,
    out_type=[
        jax.ShapeDtypeStruct((NPAD, DD), jnp.float32),
        jax.ShapeDtypeStruct((2 * NPAD, 16), jnp.float32),
        jax.ShapeDtypeStruct((2 * NPAD, 16), jnp.float32),
    ],
    scratch_types=[
        pltpu.VMEM((XBT, KX), jnp.int32),
        pltpu.VMEM((KX, DD), jnp.float32),
        pltpu.VMEM((IC, KB), jnp.int32),
        pltpu.VMEM((KB, 16), jnp.float32),
        pltpu.VMEM((128, 16), jnp.float32),
        pltpu.VMEM_SHARED((NPAD, 16), jnp.float32),
        pltpu.SemaphoreType.DMA,
    ],
)
def _sc_prep(emb_hbm, x2d_hbm, dst0_hbm, dst1_hbm,
             x0_out, hist0_out, hist1_out,
             idxn, gb0, idxe, ones_v, zero_v, h_sp, sem0):
  c = lax.axis_index("c")
  s = lax.axis_index("s")
  w = c * 16 + s

  def fill(i, carry):
    ones_v[i, :] = jnp.ones((16,), jnp.float32)
    return carry
  lax.fori_loop(0, KB, fill, 0)

  def fillz(i, carry):
    zero_v[i, :] = jnp.zeros((16,), jnp.float32)
    return carry
  lax.fori_loop(0, 128, fillz, 0)

  pltpu.sync_copy(x2d_hbm.at[pl.ds(w * XBT, XBT)], idxn)
  for q in range(XBT):
    pltpu.async_copy(emb_hbm.at[idxn.at[q]], gb0, sem0).wait()
    pltpu.sync_copy(gb0, x0_out.at[pl.ds(w * (XBT * KX) + q * KX, KX)])

  for dst_hbm, hist_out in ((dst0_hbm, hist0_out), (dst1_hbm, hist1_out)):
    for q in range(SLAB // 128):
      pltpu.sync_copy(zero_v, h_sp.at[pl.ds(s * SLAB + q * 128, 128)])
    plsc.subcore_barrier()

    def hchunk(qq, carry):
      pltpu.sync_copy(dst_hbm.at[pl.ds(w * BH + qq * IC, IC)], idxe)

      def hbody(j, carry2):
        pltpu.sync_copy(ones_v, h_sp.at[idxe.at[j]], add=True)
        return carry2
      return lax.fori_loop(0, IC, hbody, carry)
    lax.fori_loop(0, BH // IC, hchunk, 0)
    plsc.subcore_barrier()
    pltpu.sync_copy(h_sp.at[pl.ds(s * SLAB, SLAB)],
                    hist_out.at[pl.ds(c * NPAD + s * SLAB, SLAB)])
    plsc.subcore_barrier()


@functools.partial(
    pl.kernel,
    mesh=_sc_mesh,
    out_type=jax.ShapeDtypeStruct((2 * NPAD, AA), jnp.float32),
    scratch_types=[
        pltpu.VMEM((IC, KB), jnp.int32),
        pltpu.VMEM((IC, KB), jnp.int32),
        pltpu.VMEM((KB, AA), jnp.float32),
        pltpu.VMEM((KB, AA), jnp.float32),
        pltpu.VMEM_SHARED((NPAD, AA), jnp.float32),
        pltpu.SemaphoreType.DMA,
        pltpu.SemaphoreType.DMA,
        pltpu.SemaphoreType.DMA,
        pltpu.SemaphoreType.DMA,
    ],
)
def _sc_scatter(hp_hbm, srcm_hbm, dstm_hbm, s_out,
                idxs, idxd, gb0, gb1, acc_sp, sg0, sg1, ss0, ss1):
  c = lax.axis_index("c")
  s = lax.axis_index("s")
  coff = c * NPAD
  pltpu.sync_copy(hp_hbm.at[pl.ds(coff + s * SLAB, SLAB)],
                  acc_sp.at[pl.ds(s * SLAB, SLAB)])
  plsc.subcore_barrier()

  def chunk(qq, carry):
    pltpu.sync_copy(
        srcm_hbm.at[pl.ds(c * ROWS_B + s * BT + qq * IC, IC)], idxs)
    pltpu.sync_copy(dstm_hbm.at[pl.ds(s * BT + qq * IC, IC)], idxd)
    pltpu.async_copy(hp_hbm.at[idxs.at[0]], gb0, sg0)

    def body(jj, carry2):
      j0 = 2 * jj
      j1 = j0 + 1
      pltpu.make_async_copy(hp_hbm.at[idxs.at[j0]], gb0, sg0).wait()

      @pl.when(jj > 0)
      def _():
        pltpu.make_async_copy(gb1, acc_sp.at[idxd.at[j0]], ss1).wait()

      pltpu.async_copy(hp_hbm.at[idxs.at[j1]], gb1, sg1)
      pltpu.async_copy(gb0, acc_sp.at[idxd.at[j0]], ss0, add=True)
      pltpu.make_async_copy(hp_hbm.at[idxs.at[j1]], gb1, sg1).wait()
      pltpu.make_async_copy(gb0, acc_sp.at[idxd.at[j0]], ss0).wait()
      j2 = jnp.minimum(j0 + 2, IC - 1)
      pltpu.async_copy(hp_hbm.at[idxs.at[j2]], gb0, sg0)
      pltpu.async_copy(gb1, acc_sp.at[idxd.at[j1]], ss1, add=True)
      return carry2
    lax.fori_loop(0, IC // 2, body, carry)
    pltpu.make_async_copy(gb1, acc_sp.at[idxd.at[IC - 1]], ss1).wait()
    pltpu.make_async_copy(hp_hbm.at[idxs.at[IC - 1]], gb0, sg0).wait()
    return carry
  lax.fori_loop(0, BT // IC, chunk, 0)
  plsc.subcore_barrier()
  pltpu.sync_copy(acc_sp.at[pl.ds(s * SLAB, SLAB)],
                  s_out.at[pl.ds(coff + s * SLAB, SLAB)])


def _dis_block(dp_ref):
  deg = dp_ref[0, :, 0:1] + dp_ref[1, :, 0:1]
  return lax.rsqrt(1.0 + deg)


def _m1_body(x_ref, w1_ref, dp0_ref, dp1_ref, o0_ref, o1_ref):
  xb = jnp.maximum(x_ref[...], 0.0)
  h = jnp.dot(xb, w1_ref[...], preferred_element_type=jnp.float32)
  for dp_ref, o_ref in ((dp0_ref, o0_ref), (dp1_ref, o1_ref)):
    dis = _dis_block(dp_ref)
    o_ref[0, :, :] = h[:, :AA] * dis
    o_ref[1, :, :] = h[:, AA:] * dis


def _m1(x0, w1, dp0, dp1):
  return pl.pallas_call(
      _m1_body,
      grid=(GRID,),
      in_specs=[
          pl.BlockSpec((BN, DD), lambda i: (i, 0)),
          pl.BlockSpec((DD, DD), lambda i: (0, 0)),
          pl.BlockSpec((2, BN, 16), lambda i: (0, i, 0)),
          pl.BlockSpec((2, BN, 16), lambda i: (0, i, 0)),
      ],
      out_specs=[pl.BlockSpec((2, BN, AA), lambda i: (0, i, 0))] * 2,
      out_shape=[jax.ShapeDtypeStruct((2, NPAD, AA), jnp.float32)] * 2,
  )(x0, w1, dp0, dp1)


def _m2_body(s1_ref, w2_ref, b1_ref, dp_ref, o_ref):
  dis = _dis_block(dp_ref)
  a1 = jnp.concatenate([s1_ref[0], s1_ref[1]], axis=1) * dis + b1_ref[...]
  h = jnp.dot(jnp.maximum(a1, 0.0), w2_ref[...],
              preferred_element_type=jnp.float32)
  o_ref[0, :, :] = h[:, :AA] * dis
  o_ref[1, :, :] = h[:, AA:] * dis


def _m2(s1, w2, b1, dp):
  return pl.pallas_call(
      _m2_body,
      grid=(GRID,),
      in_specs=[
          pl.BlockSpec((2, BN, AA), lambda i: (0, i, 0)),
          pl.BlockSpec((DD, DD), lambda i: (0, 0)),
          pl.BlockSpec((1, DD), lambda i: (0, 0)),
          pl.BlockSpec((2, BN, 16), lambda i: (0, i, 0)),
      ],
      out_specs=pl.BlockSpec((2, BN, AA), lambda i: (0, i, 0)),
      out_shape=jax.ShapeDtypeStruct((2, NPAD, AA), jnp.float32),
  )(s1, w2, b1, dp)


def _score_body(s20_ref, s21_ref, dp0_ref, dp1_ref, b2_ref,
                wsem_ref, bsem_ref, qsem_ref, att_ref, acc_ref):
  i = pl.program_id(0)

  @pl.when(i == 0)
  def _():
    acc_ref[0, 0] = 0.0
    acc_ref[0, 1] = 0.0

  for g, (s2_ref, dp_ref) in enumerate(((s20_ref, dp0_ref),
                                        (s21_ref, dp1_ref))):
    dis = _dis_block(dp_ref)
    out = jnp.maximum(
        jnp.concatenate([s2_ref[0], s2_ref[1]], axis=1) * dis + b2_ref[...],
        0.0)
    hp = jnp.tanh(jnp.dot(out, wsem_ref[...],
                          preferred_element_type=jnp.float32) + bsem_ref[...])
    acc_ref[0, g] += jnp.sum(hp * qsem_ref[...])

  @pl.when(i == GRID - 1)
  def _():
    s0 = acc_ref[0, 0] / NN
    s1 = acc_ref[0, 1] / NN
    m = jnp.maximum(s0, s1)
    e0 = jnp.exp(s0 - m)
    e1 = jnp.exp(s1 - m)
    att_ref[0, 0] = e0 / (e0 + e1)
    att_ref[0, 1] = e1 / (e0 + e1)


def _score(s20, s21, dp0, dp1, b2, wsem, bsem, qsem):
  return pl.pallas_call(
      _score_body,
      grid=(GRID,),
      in_specs=[
          pl.BlockSpec((2, BN, AA), lambda i: (0, i, 0)),
          pl.BlockSpec((2, BN, AA), lambda i: (0, i, 0)),
          pl.BlockSpec((2, BN, 16), lambda i: (0, i, 0)),
          pl.BlockSpec((2, BN, 16), lambda i: (0, i, 0)),
          pl.BlockSpec((1, DD), lambda i: (0, 0)),
          pl.BlockSpec((DD, AA), lambda i: (0, 0)),
          pl.BlockSpec((1, AA), lambda i: (0, 0)),
          pl.BlockSpec((1, AA), lambda i: (0, 0)),
      ],
      out_specs=pl.BlockSpec(memory_space=pltpu.SMEM),
      out_shape=jax.ShapeDtypeStruct((1, 2), jnp.float32),
      scratch_shapes=[pltpu.SMEM((1, 2), jnp.float32)],
  )(s20, s21, dp0, dp1, b2, wsem, bsem, qsem)


def _final_body(att_ref, s20_ref, s21_ref, dp0_ref, dp1_ref, b2_ref, o_ref):
  outs = []
  for s2_ref, dp_ref in ((s20_ref, dp0_ref), (s21_ref, dp1_ref)):
    dis = _dis_block(dp_ref)
    outs.append(jnp.maximum(
        jnp.concatenate([s2_ref[0], s2_ref[1]], axis=1) * dis + b2_ref[...],
        0.0))
  o_ref[...] = outs[0] * att_ref[0, 0] + outs[1] * att_ref[0, 1]


def _final(att, s20, s21, dp0, dp1, b2):
  return pl.pallas_call(
      _final_body,
      grid=(GRID,),
      in_specs=[
          pl.BlockSpec(memory_space=pltpu.SMEM),
          pl.BlockSpec((2, BN, AA), lambda i: (0, i, 0)),
          pl.BlockSpec((2, BN, AA), lambda i: (0, i, 0)),
          pl.BlockSpec((2, BN, 16), lambda i: (0, i, 0)),
          pl.BlockSpec((2, BN, 16), lambda i: (0, i, 0)),
          pl.BlockSpec((1, DD), lambda i: (0, 0)),
      ],
      out_specs=pl.BlockSpec((BN, DD), lambda i: (i, 0)),
      out_shape=jax.ShapeDtypeStruct((NN, DD), jnp.float32),
  )(att, s20, s21, dp0, dp1, b2)


def _edge_arrays(ei):
  src = ei[0].astype(jnp.int32)
  dst = ei[1].astype(jnp.int32)
  pad_s = jnp.zeros((EPAD - EE,), jnp.int32)
  pad_d = jnp.full((EPAD - EE,), NN, jnp.int32)
  src_p = jnp.concatenate([src, pad_s]).reshape(ROWS_B, KB)
  dst_p = jnp.concatenate([dst, pad_d]).reshape(ROWS_B, KB)
  srcm = jnp.concatenate([src_p, src_p + NPAD], axis=0)
  return srcm, dst_p


def kernel(x, edge_index_0, edge_index_1, emb_table, W1, b1, W2, b2,
           Wsem, bsem, qsem):
  x_i = x.astype(jnp.int32)
  x2d = jnp.concatenate(
      [x_i, jnp.zeros((NPAD - NN,), jnp.int32)]).reshape(XROWS, KX)
  src0m, dst0m = _edge_arrays(edge_index_0)
  src1m, dst1m = _edge_arrays(edge_index_1)

  x0, hist0, hist1 = _sc_prep(emb_table, x2d, dst0m, dst1m)
  dp0 = hist0.reshape(2, NPAD, 16)
  dp1 = hist1.reshape(2, NPAD, 16)

  h1p0, h1p1 = _m1(x0, W1, dp0, dp1)

  s10 = _sc_scatter(h1p0.reshape(2 * NPAD, AA), src0m, dst0m).reshape(
      2, NPAD, AA)
  s11 = _sc_scatter(h1p1.reshape(2 * NPAD, AA), src1m, dst1m).reshape(
      2, NPAD, AA)

  b1r = b1.reshape(1, DD)
  h2p0 = _m2(s10, W2, b1r, dp0)
  h2p1 = _m2(s11, W2, b1r, dp1)

  s20 = _sc_scatter(h2p0.reshape(2 * NPAD, AA), src0m, dst0m).reshape(
      2, NPAD, AA)
  s21 = _sc_scatter(h2p1.reshape(2 * NPAD, AA), src1m, dst1m).reshape(
      2, NPAD, AA)

  b2r = b2.reshape(1, DD)
  att = _score(s20, s21, dp0, dp1, b2r, Wsem, bsem, qsem)
  h_emb = _final(att, s20, s21, dp0, dp1, b2r)
  return (h_emb, att)


# speedup vs baseline: 1.0018x; 1.0018x over previous
import functools

import jax
import jax.numpy as jnp
from jax import lax
from jax.experimental import pallas as pl
from jax.experimental.pallas import tpu as pltpu
from jax.experimental.pallas import tpu_sc as plsc

NN = 10000
EE = 160000
DD = 256
AA = 128
NPAD = 10240
EPAD = 163840
KB = 64
ROWS_B = EPAD // KB
BT = ROWS_B // 16
BH = ROWS_B // 32
IC = 16
SLAB = NPAD // 16
KX = 40
XROWS = NPAD // KX
XBT = XROWS // 32
BN = 400
GRID = NN // BN

_sc_mesh = plsc.VectorSubcoreMesh(core_axis_name="c", subcore_axis_name="s")


@functools.partial(
    pl.kernel,
    mesh=_sc_mesh,
    out_type=[
        jax.ShapeDtypeStruct((NPAD, DD), jnp.float32),
        jax.ShapeDtypeStruct((2 * NPAD, 16), jnp.float32),
        jax.ShapeDtypeStruct((2 * NPAD, 16), jnp.float32),
    ],
    scratch_types=[
        pltpu.VMEM((XBT, KX), jnp.int32),
        pltpu.VMEM((KX, DD), jnp.float32),
        pltpu.VMEM((IC, KB), jnp.int32),
        pltpu.VMEM((KB, 16), jnp.float32),
        pltpu.VMEM((128, 16), jnp.float32),
        pltpu.VMEM_SHARED((NPAD, 16), jnp.float32),
        pltpu.SemaphoreType.DMA,
    ],
)
def _sc_prep(emb_hbm, x2d_hbm, dst0_hbm, dst1_hbm,
             x0_out, hist0_out, hist1_out,
             idxn, gb0, idxe, ones_v, zero_v, h_sp, sem0):
  c = lax.axis_index("c")
  s = lax.axis_index("s")
  w = c * 16 + s

  def fill(i, carry):
    ones_v[i, :] = jnp.ones((16,), jnp.float32)
    return carry
  lax.fori_loop(0, KB, fill, 0)

  def fillz(i, carry):
    zero_v[i, :] = jnp.zeros((16,), jnp.float32)
    return carry
  lax.fori_loop(0, 128, fillz, 0)

  pltpu.sync_copy(x2d_hbm.at[pl.ds(w * XBT, XBT)], idxn)
  for q in range(XBT):
    pltpu.async_copy(emb_hbm.at[idxn.at[q]], gb0, sem0).wait()
    pltpu.sync_copy(gb0, x0_out.at[pl.ds(w * (XBT * KX) + q * KX, KX)])

  for dst_hbm, hist_out in ((dst0_hbm, hist0_out), (dst1_hbm, hist1_out)):
    for q in range(SLAB // 128):
      pltpu.sync_copy(zero_v, h_sp.at[pl.ds(s * SLAB + q * 128, 128)])
    plsc.subcore_barrier()

    def hchunk(qq, carry):
      pltpu.sync_copy(dst_hbm.at[pl.ds(w * BH + qq * IC, IC)], idxe)

      def hbody(j, carry2):
        pltpu.sync_copy(ones_v, h_sp.at[idxe.at[j]], add=True)
        return carry2
      return lax.fori_loop(0, IC, hbody, carry)
    lax.fori_loop(0, BH // IC, hchunk, 0)
    plsc.subcore_barrier()
    pltpu.sync_copy(h_sp.at[pl.ds(s * SLAB, SLAB)],
                    hist_out.at[pl.ds(c * NPAD + s * SLAB, SLAB)])
    plsc.subcore_barrier()


@functools.partial(
    pl.kernel,
    mesh=_sc_mesh,
    out_type=[
        jax.ShapeDtypeStruct((2 * NPAD, AA), jnp.float32),
        jax.ShapeDtypeStruct((2 * NPAD, AA), jnp.float32),
    ],
    scratch_types=[
        pltpu.VMEM((IC, KB), jnp.int32),
        pltpu.VMEM((IC, KB), jnp.int32),
        pltpu.VMEM((KB, AA), jnp.float32),
        pltpu.VMEM((KB, AA), jnp.float32),
        pltpu.VMEM_SHARED((NPAD, AA), jnp.float32),
        pltpu.SemaphoreType.DMA,
        pltpu.SemaphoreType.DMA,
    ],
)
def _sc_scatter2(hp0_hbm, hp1_hbm, srcm0_hbm, dstm0_hbm, srcm1_hbm,
                 dstm1_hbm, s0_out, s1_out,
                 idxs, idxd, gb0, gb1, acc_sp, sem0, sem1):
  c = lax.axis_index("c")
  s = lax.axis_index("s")
  coff = c * NPAD
  for hp_hbm, srcm_hbm, dstm_hbm, s_out in (
      (hp0_hbm, srcm0_hbm, dstm0_hbm, s0_out),
      (hp1_hbm, srcm1_hbm, dstm1_hbm, s1_out)):
    pltpu.sync_copy(hp_hbm.at[pl.ds(coff + s * SLAB, SLAB)],
                    acc_sp.at[pl.ds(s * SLAB, SLAB)])
    plsc.subcore_barrier()

    def chunk(qq, carry):
      pltpu.sync_copy(
          srcm_hbm.at[pl.ds(c * ROWS_B + s * BT + qq * IC, IC)], idxs)
      pltpu.sync_copy(dstm_hbm.at[pl.ds(s * BT + qq * IC, IC)], idxd)

      def body(jj, carry2):
        j0 = 2 * jj
        d0 = pltpu.async_copy(hp_hbm.at[idxs.at[j0]], gb0, sem0)
        d1 = pltpu.async_copy(hp_hbm.at[idxs.at[j0 + 1]], gb1, sem1)
        d0.wait()
        pltpu.sync_copy(gb0, acc_sp.at[idxd.at[j0]], add=True)
        d1.wait()
        pltpu.sync_copy(gb1, acc_sp.at[idxd.at[j0 + 1]], add=True)
        return carry2
      return lax.fori_loop(0, IC // 2, body, carry)
    lax.fori_loop(0, BT // IC, chunk, 0)
    plsc.subcore_barrier()
    pltpu.sync_copy(acc_sp.at[pl.ds(s * SLAB, SLAB)],
                    s_out.at[pl.ds(coff + s * SLAB, SLAB)])
    plsc.subcore_barrier()


def _dis_block(dp_ref):
  deg = dp_ref[0, :, 0:1] + dp_ref[1, :, 0:1]
  return lax.rsqrt(1.0 + deg)


def _m1_body(x_ref, w1_ref, dp0_ref, dp1_ref, o0_ref, o1_ref):
  xb = jnp.maximum(x_ref[...], 0.0)
  h = jnp.dot(xb, w1_ref[...], preferred_element_type=jnp.float32)
  for dp_ref, o_ref in ((dp0_ref, o0_ref), (dp1_ref, o1_ref)):
    dis = _dis_block(dp_ref)
    o_ref[0, :, :] = h[:, :AA] * dis
    o_ref[1, :, :] = h[:, AA:] * dis


def _m1(x0, w1, dp0, dp1):
  return pl.pallas_call(
      _m1_body,
      grid=(GRID,),
      in_specs=[
          pl.BlockSpec((BN, DD), lambda i: (i, 0)),
          pl.BlockSpec((DD, DD), lambda i: (0, 0)),
          pl.BlockSpec((2, BN, 16), lambda i: (0, i, 0)),
          pl.BlockSpec((2, BN, 16), lambda i: (0, i, 0)),
      ],
      out_specs=[pl.BlockSpec((2, BN, AA), lambda i: (0, i, 0))] * 2,
      out_shape=[jax.ShapeDtypeStruct((2, NPAD, AA), jnp.float32)] * 2,
  )(x0, w1, dp0, dp1)


def _m2_body(s10_ref, s11_ref, w2_ref, b1_ref, dp0_ref, dp1_ref,
             o0_ref, o1_ref):
  for s1_ref, dp_ref, o_ref in ((s10_ref, dp0_ref, o0_ref),
                                (s11_ref, dp1_ref, o1_ref)):
    dis = _dis_block(dp_ref)
    a1 = jnp.concatenate([s1_ref[0], s1_ref[1]], axis=1) * dis + b1_ref[...]
    h = jnp.dot(jnp.maximum(a1, 0.0), w2_ref[...],
                preferred_element_type=jnp.float32)
    o_ref[0, :, :] = h[:, :AA] * dis
    o_ref[1, :, :] = h[:, AA:] * dis


def _m2(s10, s11, w2, b1, dp0, dp1):
  return pl.pallas_call(
      _m2_body,
      grid=(GRID,),
      in_specs=[
          pl.BlockSpec((2, BN, AA), lambda i: (0, i, 0)),
          pl.BlockSpec((2, BN, AA), lambda i: (0, i, 0)),
          pl.BlockSpec((DD, DD), lambda i: (0, 0)),
          pl.BlockSpec((1, DD), lambda i: (0, 0)),
          pl.BlockSpec((2, BN, 16), lambda i: (0, i, 0)),
          pl.BlockSpec((2, BN, 16), lambda i: (0, i, 0)),
      ],
      out_specs=[pl.BlockSpec((2, BN, AA), lambda i: (0, i, 0))] * 2,
      out_shape=[jax.ShapeDtypeStruct((2, NPAD, AA), jnp.float32)] * 2,
  )(s10, s11, w2, b1, dp0, dp1)


def _score_body(s20_ref, s21_ref, dp0_ref, dp1_ref, b2_ref,
                wsem_ref, bsem_ref, qsem_ref, att_ref, acc_ref):
  i = pl.program_id(0)

  @pl.when(i == 0)
  def _():
    acc_ref[0, 0] = 0.0
    acc_ref[0, 1] = 0.0

  for g, (s2_ref, dp_ref) in enumerate(((s20_ref, dp0_ref),
                                        (s21_ref, dp1_ref))):
    dis = _dis_block(dp_ref)
    out = jnp.maximum(
        jnp.concatenate([s2_ref[0], s2_ref[1]], axis=1) * dis + b2_ref[...],
        0.0)
    hp = jnp.tanh(jnp.dot(out, wsem_ref[...],
                          preferred_element_type=jnp.float32) + bsem_ref[...])
    acc_ref[0, g] += jnp.sum(hp * qsem_ref[...])

  @pl.when(i == GRID - 1)
  def _():
    s0 = acc_ref[0, 0] / NN
    s1 = acc_ref[0, 1] / NN
    m = jnp.maximum(s0, s1)
    e0 = jnp.exp(s0 - m)
    e1 = jnp.exp(s1 - m)
    att_ref[0, 0] = e0 / (e0 + e1)
    att_ref[0, 1] = e1 / (e0 + e1)


def _score(s20, s21, dp0, dp1, b2, wsem, bsem, qsem):
  return pl.pallas_call(
      _score_body,
      grid=(GRID,),
      in_specs=[
          pl.BlockSpec((2, BN, AA), lambda i: (0, i, 0)),
          pl.BlockSpec((2, BN, AA), lambda i: (0, i, 0)),
          pl.BlockSpec((2, BN, 16), lambda i: (0, i, 0)),
          pl.BlockSpec((2, BN, 16), lambda i: (0, i, 0)),
          pl.BlockSpec((1, DD), lambda i: (0, 0)),
          pl.BlockSpec((DD, AA), lambda i: (0, 0)),
          pl.BlockSpec((1, AA), lambda i: (0, 0)),
          pl.BlockSpec((1, AA), lambda i: (0, 0)),
      ],
      out_specs=pl.BlockSpec(memory_space=pltpu.SMEM),
      out_shape=jax.ShapeDtypeStruct((1, 2), jnp.float32),
      scratch_shapes=[pltpu.SMEM((1, 2), jnp.float32)],
  )(s20, s21, dp0, dp1, b2, wsem, bsem, qsem)


def _final_body(att_ref, s20_ref, s21_ref, dp0_ref, dp1_ref, b2_ref, o_ref):
  outs = []
  for s2_ref, dp_ref in ((s20_ref, dp0_ref), (s21_ref, dp1_ref)):
    dis = _dis_block(dp_ref)
    outs.append(jnp.maximum(
        jnp.concatenate([s2_ref[0], s2_ref[1]], axis=1) * dis + b2_ref[...],
        0.0))
  o_ref[...] = outs[0] * att_ref[0, 0] + outs[1] * att_ref[0, 1]


def _final(att, s20, s21, dp0, dp1, b2):
  return pl.pallas_call(
      _final_body,
      grid=(GRID,),
      in_specs=[
          pl.BlockSpec(memory_space=pltpu.SMEM),
          pl.BlockSpec((2, BN, AA), lambda i: (0, i, 0)),
          pl.BlockSpec((2, BN, AA), lambda i: (0, i, 0)),
          pl.BlockSpec((2, BN, 16), lambda i: (0, i, 0)),
          pl.BlockSpec((2, BN, 16), lambda i: (0, i, 0)),
          pl.BlockSpec((1, DD), lambda i: (0, 0)),
      ],
      out_specs=pl.BlockSpec((BN, DD), lambda i: (i, 0)),
      out_shape=jax.ShapeDtypeStruct((NN, DD), jnp.float32),
  )(att, s20, s21, dp0, dp1, b2)


def _edge_arrays(ei):
  src = ei[0].astype(jnp.int32)
  dst = ei[1].astype(jnp.int32)
  pad_s = jnp.zeros((EPAD - EE,), jnp.int32)
  pad_d = jnp.full((EPAD - EE,), NN, jnp.int32)
  src_p = jnp.concatenate([src, pad_s]).reshape(ROWS_B, KB)
  dst_p = jnp.concatenate([dst, pad_d]).reshape(ROWS_B, KB)
  srcm = jnp.concatenate([src_p, src_p + NPAD], axis=0)
  return srcm, dst_p


def kernel(x, edge_index_0, edge_index_1, emb_table, W1, b1, W2, b2,
           Wsem, bsem, qsem):
  x_i = x.astype(jnp.int32)
  x2d = jnp.concatenate(
      [x_i, jnp.zeros((NPAD - NN,), jnp.int32)]).reshape(XROWS, KX)
  src0m, dst0m = _edge_arrays(edge_index_0)
  src1m, dst1m = _edge_arrays(edge_index_1)

  x0, hist0, hist1 = _sc_prep(emb_table, x2d, dst0m, dst1m)
  dp0 = hist0.reshape(2, NPAD, 16)
  dp1 = hist1.reshape(2, NPAD, 16)

  h1p0, h1p1 = _m1(x0, W1, dp0, dp1)

  s10f, s11f = _sc_scatter2(h1p0.reshape(2 * NPAD, AA),
                            h1p1.reshape(2 * NPAD, AA),
                            src0m, dst0m, src1m, dst1m)
  s10 = s10f.reshape(2, NPAD, AA)
  s11 = s11f.reshape(2, NPAD, AA)

  b1r = b1.reshape(1, DD)
  h2p0, h2p1 = _m2(s10, s11, W2, b1r, dp0, dp1)

  s20f, s21f = _sc_scatter2(h2p0.reshape(2 * NPAD, AA),
                            h2p1.reshape(2 * NPAD, AA),
                            src0m, dst0m, src1m, dst1m)
  s20 = s20f.reshape(2, NPAD, AA)
  s21 = s21f.reshape(2, NPAD, AA)

  b2r = b2.reshape(1, DD)
  att = _score(s20, s21, dp0, dp1, b2r, Wsem, bsem, qsem)
  h_emb = _final(att, s20, s21, dp0, dp1, b2r)
  return (h_emb, att)


# speedup vs baseline: 1.0245x; 1.0227x over previous
import functools

import jax
import jax.numpy as jnp
from jax import lax
from jax.experimental import pallas as pl
from jax.experimental.pallas import tpu as pltpu
from jax.experimental.pallas import tpu_sc as plsc

NN = 10000
EE = 160000
DD = 256
AA = 128
NPAD = 10240
EPAD = 163840
KB = 64
ROWS_B = EPAD // KB
BT = ROWS_B // 16
BH = ROWS_B // 32
IC = 16
SLAB = NPAD // 16
KX = 40
XROWS = NPAD // KX
XBT = XROWS // 32
BN = 400
GRID = NN // BN

_sc_mesh = plsc.VectorSubcoreMesh(core_axis_name="c", subcore_axis_name="s")


@functools.partial(
    pl.kernel,
    mesh=_sc_mesh,
    out_type=[
        jax.ShapeDtypeStruct((NPAD, DD), jnp.float32),
        jax.ShapeDtypeStruct((2 * NPAD, 16), jnp.float32),
        jax.ShapeDtypeStruct((2 * NPAD, 16), jnp.float32),
    ],
    scratch_types=[
        pltpu.VMEM((XBT, KX), jnp.int32),
        pltpu.VMEM((KX, DD), jnp.float32),
        pltpu.VMEM((IC, KB), jnp.int32),
        pltpu.VMEM((KB, 16), jnp.float32),
        pltpu.VMEM((128, 16), jnp.float32),
        pltpu.VMEM_SHARED((NPAD, 16), jnp.float32),
        pltpu.SemaphoreType.DMA,
    ],
)
def _sc_prep(emb_hbm, x2d_hbm, dst0_hbm, dst1_hbm,
             x0_out, hist0_out, hist1_out,
             idxn, gb0, idxe, ones_v, zero_v, h_sp, sem0):
  c = lax.axis_index("c")
  s = lax.axis_index("s")
  w = c * 16 + s

  def fill(i, carry):
    ones_v[i, :] = jnp.ones((16,), jnp.float32)
    return carry
  lax.fori_loop(0, KB, fill, 0)

  def fillz(i, carry):
    zero_v[i, :] = jnp.zeros((16,), jnp.float32)
    return carry
  lax.fori_loop(0, 128, fillz, 0)

  pltpu.sync_copy(x2d_hbm.at[pl.ds(w * XBT, XBT)], idxn)
  for q in range(XBT):
    pltpu.async_copy(emb_hbm.at[idxn.at[q]], gb0, sem0).wait()
    pltpu.sync_copy(gb0, x0_out.at[pl.ds(w * (XBT * KX) + q * KX, KX)])

  for dst_hbm, hist_out in ((dst0_hbm, hist0_out), (dst1_hbm, hist1_out)):
    for q in range(SLAB // 128):
      pltpu.sync_copy(zero_v, h_sp.at[pl.ds(s * SLAB + q * 128, 128)])
    plsc.subcore_barrier()

    def hchunk(qq, carry):
      pltpu.sync_copy(dst_hbm.at[pl.ds(w * BH + qq * IC, IC)], idxe)

      def hbody(j, carry2):
        pltpu.sync_copy(ones_v, h_sp.at[idxe.at[j]], add=True)
        return carry2
      return lax.fori_loop(0, IC, hbody, carry)
    lax.fori_loop(0, BH // IC, hchunk, 0)
    plsc.subcore_barrier()
    pltpu.sync_copy(h_sp.at[pl.ds(s * SLAB, SLAB)],
                    hist_out.at[pl.ds(c * NPAD + s * SLAB, SLAB)])
    plsc.subcore_barrier()


@functools.partial(
    pl.kernel,
    mesh=_sc_mesh,
    out_type=jax.ShapeDtypeStruct((2 * NPAD, AA), jnp.float32),
    scratch_types=[
        pltpu.VMEM((IC, KB), jnp.int32),
        pltpu.VMEM((IC, KB), jnp.int32),
        pltpu.VMEM((KB, AA), jnp.float32),
        pltpu.VMEM((KB, AA), jnp.float32),
        pltpu.VMEM_SHARED((NPAD, AA), jnp.float32),
        pltpu.SemaphoreType.DMA,
        pltpu.SemaphoreType.DMA,
    ],
)
def _sc_scatter(hp_hbm, srcm_hbm, dstm_hbm, s_out,
                idxs, idxd, gb0, gb1, acc_sp, sem0, sem1):
  c = lax.axis_index("c")
  s = lax.axis_index("s")
  coff = c * NPAD
  pltpu.sync_copy(hp_hbm.at[pl.ds(coff + s * SLAB, SLAB)],
                  acc_sp.at[pl.ds(s * SLAB, SLAB)])
  plsc.subcore_barrier()

  def chunk(qq, carry):
    pltpu.sync_copy(
        srcm_hbm.at[pl.ds(c * ROWS_B + s * BT + qq * IC, IC)], idxs)
    pltpu.sync_copy(dstm_hbm.at[pl.ds(s * BT + qq * IC, IC)], idxd)

    def body(jj, carry2):
      j0 = 2 * jj
      d0 = pltpu.async_copy(hp_hbm.at[idxs.at[j0]], gb0, sem0)
      d1 = pltpu.async_copy(hp_hbm.at[idxs.at[j0 + 1]], gb1, sem1)
      d0.wait()
      pltpu.sync_copy(gb0, acc_sp.at[idxd.at[j0]], add=True)
      d1.wait()
      pltpu.sync_copy(gb1, acc_sp.at[idxd.at[j0 + 1]], add=True)
      return carry2
    return lax.fori_loop(0, IC // 2, body, carry)
  lax.fori_loop(0, BT // IC, chunk, 0)
  plsc.subcore_barrier()
  pltpu.sync_copy(acc_sp.at[pl.ds(s * SLAB, SLAB)],
                  s_out.at[pl.ds(coff + s * SLAB, SLAB)])


def _dis_block(dp_ref):
  deg = dp_ref[0, :, 0:1] + dp_ref[1, :, 0:1]
  return lax.rsqrt(1.0 + deg)


def _m1_body(x_ref, w1_ref, dp0_ref, dp1_ref, o0_ref, o1_ref):
  xb = jnp.maximum(x_ref[...], 0.0)
  h = jnp.dot(xb, w1_ref[...], preferred_element_type=jnp.float32)
  for dp_ref, o_ref in ((dp0_ref, o0_ref), (dp1_ref, o1_ref)):
    dis = _dis_block(dp_ref)
    o_ref[0, :, :] = h[:, :AA] * dis
    o_ref[1, :, :] = h[:, AA:] * dis


def _m1(x0, w1, dp0, dp1):
  return pl.pallas_call(
      _m1_body,
      grid=(GRID,),
      in_specs=[
          pl.BlockSpec((BN, DD), lambda i: (i, 0)),
          pl.BlockSpec((DD, DD), lambda i: (0, 0)),
          pl.BlockSpec((2, BN, 16), lambda i: (0, i, 0)),
          pl.BlockSpec((2, BN, 16), lambda i: (0, i, 0)),
      ],
      out_specs=[pl.BlockSpec((2, BN, AA), lambda i: (0, i, 0))] * 2,
      out_shape=[jax.ShapeDtypeStruct((2, NPAD, AA), jnp.float32)] * 2,
  )(x0, w1, dp0, dp1)


def _m2_body(s1_ref, w2_ref, b1_ref, dp_ref, o_ref):
  dis = _dis_block(dp_ref)
  a1 = jnp.concatenate([s1_ref[0], s1_ref[1]], axis=1) * dis + b1_ref[...]
  h = jnp.dot(jnp.maximum(a1, 0.0), w2_ref[...],
              preferred_element_type=jnp.float32)
  o_ref[0, :, :] = h[:, :AA] * dis
  o_ref[1, :, :] = h[:, AA:] * dis


def _m2(s1, w2, b1, dp):
  return pl.pallas_call(
      _m2_body,
      grid=(GRID,),
      in_specs=[
          pl.BlockSpec((2, BN, AA), lambda i: (0, i, 0)),
          pl.BlockSpec((DD, DD), lambda i: (0, 0)),
          pl.BlockSpec((1, DD), lambda i: (0, 0)),
          pl.BlockSpec((2, BN, 16), lambda i: (0, i, 0)),
      ],
      out_specs=pl.BlockSpec((2, BN, AA), lambda i: (0, i, 0)),
      out_shape=jax.ShapeDtypeStruct((2, NPAD, AA), jnp.float32),
  )(s1, w2, b1, dp)


def _score_body(s20_ref, s21_ref, dp0_ref, dp1_ref, b2_ref,
                wsem_ref, bsem_ref, qsem_ref, att_ref, acc_ref):
  i = pl.program_id(0)

  @pl.when(i == 0)
  def _():
    acc_ref[0, 0] = 0.0
    acc_ref[0, 1] = 0.0

  for g, (s2_ref, dp_ref) in enumerate(((s20_ref, dp0_ref),
                                        (s21_ref, dp1_ref))):
    dis = _dis_block(dp_ref)
    out = jnp.maximum(
        jnp.concatenate([s2_ref[0], s2_ref[1]], axis=1) * dis + b2_ref[...],
        0.0)
    hp = jnp.tanh(jnp.dot(out, wsem_ref[...],
                          preferred_element_type=jnp.float32) + bsem_ref[...])
    acc_ref[0, g] += jnp.sum(hp * qsem_ref[...])

  @pl.when(i == GRID - 1)
  def _():
    s0 = acc_ref[0, 0] / NN
    s1 = acc_ref[0, 1] / NN
    m = jnp.maximum(s0, s1)
    e0 = jnp.exp(s0 - m)
    e1 = jnp.exp(s1 - m)
    att_ref[0, 0] = e0 / (e0 + e1)
    att_ref[0, 1] = e1 / (e0 + e1)


def _score(s20, s21, dp0, dp1, b2, wsem, bsem, qsem):
  return pl.pallas_call(
      _score_body,
      grid=(GRID,),
      in_specs=[
          pl.BlockSpec((2, BN, AA), lambda i: (0, i, 0)),
          pl.BlockSpec((2, BN, AA), lambda i: (0, i, 0)),
          pl.BlockSpec((2, BN, 16), lambda i: (0, i, 0)),
          pl.BlockSpec((2, BN, 16), lambda i: (0, i, 0)),
          pl.BlockSpec((1, DD), lambda i: (0, 0)),
          pl.BlockSpec((DD, AA), lambda i: (0, 0)),
          pl.BlockSpec((1, AA), lambda i: (0, 0)),
          pl.BlockSpec((1, AA), lambda i: (0, 0)),
      ],
      out_specs=pl.BlockSpec(memory_space=pltpu.SMEM),
      out_shape=jax.ShapeDtypeStruct((1, 2), jnp.float32),
      scratch_shapes=[pltpu.SMEM((1, 2), jnp.float32)],
  )(s20, s21, dp0, dp1, b2, wsem, bsem, qsem)


def _final_body(att_ref, s20_ref, s21_ref, dp0_ref, dp1_ref, b2_ref, o_ref):
  outs = []
  for s2_ref, dp_ref in ((s20_ref, dp0_ref), (s21_ref, dp1_ref)):
    dis = _dis_block(dp_ref)
    outs.append(jnp.maximum(
        jnp.concatenate([s2_ref[0], s2_ref[1]], axis=1) * dis + b2_ref[...],
        0.0))
  o_ref[...] = outs[0] * att_ref[0, 0] + outs[1] * att_ref[0, 1]


def _final(att, s20, s21, dp0, dp1, b2):
  return pl.pallas_call(
      _final_body,
      grid=(GRID,),
      in_specs=[
          pl.BlockSpec(memory_space=pltpu.SMEM),
          pl.BlockSpec((2, BN, AA), lambda i: (0, i, 0)),
          pl.BlockSpec((2, BN, AA), lambda i: (0, i, 0)),
          pl.BlockSpec((2, BN, 16), lambda i: (0, i, 0)),
          pl.BlockSpec((2, BN, 16), lambda i: (0, i, 0)),
          pl.BlockSpec((1, DD), lambda i: (0, 0)),
      ],
      out_specs=pl.BlockSpec((BN, DD), lambda i: (i, 0)),
      out_shape=jax.ShapeDtypeStruct((NN, DD), jnp.float32),
  )(att, s20, s21, dp0, dp1, b2)


def _edge_arrays(ei):
  src = ei[0].astype(jnp.int32)
  dst = ei[1].astype(jnp.int32)
  pad_s = jnp.zeros((EPAD - EE,), jnp.int32)
  pad_d = jnp.full((EPAD - EE,), NN, jnp.int32)
  src_p = jnp.concatenate([src, pad_s]).reshape(ROWS_B, KB)
  dst_p = jnp.concatenate([dst, pad_d]).reshape(ROWS_B, KB)
  srcm = jnp.concatenate([src_p, src_p + NPAD], axis=0)
  return srcm, dst_p


def kernel(x, edge_index_0, edge_index_1, emb_table, W1, b1, W2, b2,
           Wsem, bsem, qsem):
  x_i = x.astype(jnp.int32)
  x2d = jnp.concatenate(
      [x_i, jnp.zeros((NPAD - NN,), jnp.int32)]).reshape(XROWS, KX)
  src0m, dst0m = _edge_arrays(edge_index_0)
  src1m, dst1m = _edge_arrays(edge_index_1)

  x0, hist0, hist1 = _sc_prep(emb_table, x2d, dst0m, dst1m)
  dp0 = hist0.reshape(2, NPAD, 16)
  dp1 = hist1.reshape(2, NPAD, 16)

  h1p0, h1p1 = _m1(x0, W1, dp0, dp1)

  s10 = _sc_scatter(h1p0.reshape(2 * NPAD, AA), src0m, dst0m).reshape(
      2, NPAD, AA)
  s11 = _sc_scatter(h1p1.reshape(2 * NPAD, AA), src1m, dst1m).reshape(
      2, NPAD, AA)

  b1r = b1.reshape(1, DD)
  h2p0 = _m2(s10, W2, b1r, dp0)
  h2p1 = _m2(s11, W2, b1r, dp1)

  s20 = _sc_scatter(h2p0.reshape(2 * NPAD, AA), src0m, dst0m).reshape(
      2, NPAD, AA)
  s21 = _sc_scatter(h2p1.reshape(2 * NPAD, AA), src1m, dst1m).reshape(
      2, NPAD, AA)

  b2r = b2.reshape(1, DD)
  att = _score(s20, s21, dp0, dp1, b2r, Wsem, bsem, qsem)
  h_emb = _final(att, s20, s21, dp0, dp1, b2r)
  return (h_emb, att)


# speedup vs baseline: 1.0540x; 1.0287x over previous
import functools

import jax
import jax.numpy as jnp
from jax import lax
from jax.experimental import pallas as pl
from jax.experimental.pallas import tpu as pltpu
from jax.experimental.pallas import tpu_sc as plsc

NN = 10000
EE = 160000
DD = 256
AA = 128
NPAD = 10240
EPAD = 163840
KB = 64
ROWS_B = EPAD // KB
BT = ROWS_B // 16
BH = ROWS_B // 32
IC = 16
SLAB = NPAD // 16
KX = 40
XROWS = NPAD // KX
XBT = XROWS // 32
BN = 400
GRID = NN // BN

_sc_mesh = plsc.VectorSubcoreMesh(core_axis_name="c", subcore_axis_name="s")


@functools.partial(
    pl.kernel,
    mesh=_sc_mesh,
    out_type=[
        jax.ShapeDtypeStruct((NPAD, DD), jnp.float32),
        jax.ShapeDtypeStruct((2 * NPAD, 16), jnp.float32),
        jax.ShapeDtypeStruct((2 * NPAD, 16), jnp.float32),
    ],
    scratch_types=[
        pltpu.VMEM((XBT, KX), jnp.int32),
        pltpu.VMEM((KX, DD), jnp.float32),
        pltpu.VMEM((IC, KB), jnp.int32),
        pltpu.VMEM((KB, 16), jnp.float32),
        pltpu.VMEM((128, 16), jnp.float32),
        pltpu.VMEM_SHARED((NPAD, 16), jnp.float32),
        pltpu.SemaphoreType.DMA,
    ],
)
def _sc_prep(emb_hbm, x2d_hbm, dst0_hbm, dst1_hbm,
             x0_out, hist0_out, hist1_out,
             idxn, gb0, idxe, ones_v, zero_v, h_sp, sem0):
  c = lax.axis_index("c")
  s = lax.axis_index("s")
  w = c * 16 + s

  def fill(i, carry):
    ones_v[i, :] = jnp.ones((16,), jnp.float32)
    return carry
  lax.fori_loop(0, KB, fill, 0)

  def fillz(i, carry):
    zero_v[i, :] = jnp.zeros((16,), jnp.float32)
    return carry
  lax.fori_loop(0, 128, fillz, 0)

  pltpu.sync_copy(x2d_hbm.at[pl.ds(w * XBT, XBT)], idxn)
  for q in range(XBT):
    pltpu.async_copy(emb_hbm.at[idxn.at[q]], gb0, sem0).wait()
    pltpu.sync_copy(gb0, x0_out.at[pl.ds(w * (XBT * KX) + q * KX, KX)])

  for dst_hbm, hist_out in ((dst0_hbm, hist0_out), (dst1_hbm, hist1_out)):
    for q in range(SLAB // 128):
      pltpu.sync_copy(zero_v, h_sp.at[pl.ds(s * SLAB + q * 128, 128)])
    plsc.subcore_barrier()

    def hchunk(qq, carry):
      pltpu.sync_copy(dst_hbm.at[pl.ds(w * BH + qq * IC, IC)], idxe)

      def hbody(j, carry2):
        pltpu.sync_copy(ones_v, h_sp.at[idxe.at[j]], add=True)
        return carry2
      return lax.fori_loop(0, IC, hbody, carry)
    lax.fori_loop(0, BH // IC, hchunk, 0)
    plsc.subcore_barrier()
    pltpu.sync_copy(h_sp.at[pl.ds(s * SLAB, SLAB)],
                    hist_out.at[pl.ds(c * NPAD + s * SLAB, SLAB)])
    plsc.subcore_barrier()


@functools.partial(
    pl.kernel,
    mesh=_sc_mesh,
    out_type=[
        jax.ShapeDtypeStruct((2 * NPAD, AA), jnp.float32),
        jax.ShapeDtypeStruct((2 * NPAD, AA), jnp.float32),
    ],
    scratch_types=[
        pltpu.VMEM((IC, KB), jnp.int32),
        pltpu.VMEM((IC, KB), jnp.int32),
        pltpu.VMEM((KB, AA), jnp.float32),
        pltpu.VMEM((KB, AA), jnp.float32),
        pltpu.VMEM_SHARED((NPAD, AA), jnp.float32),
        pltpu.SemaphoreType.DMA,
        pltpu.SemaphoreType.DMA,
    ],
)
def _sc_scatter2(hp0_hbm, hp1_hbm, srcm0_hbm, dstm0_hbm, srcm1_hbm,
                 dstm1_hbm, s0_out, s1_out,
                 idxs, idxd, gb0, gb1, acc_sp, sem0, sem1):
  c = lax.axis_index("c")
  s = lax.axis_index("s")
  coff = c * NPAD
  for hp_hbm, srcm_hbm, dstm_hbm, s_out in (
      (hp0_hbm, srcm0_hbm, dstm0_hbm, s0_out),
      (hp1_hbm, srcm1_hbm, dstm1_hbm, s1_out)):
    pltpu.sync_copy(hp_hbm.at[pl.ds(coff + s * SLAB, SLAB)],
                    acc_sp.at[pl.ds(s * SLAB, SLAB)])
    plsc.subcore_barrier()

    def chunk(qq, carry):
      pltpu.sync_copy(
          srcm_hbm.at[pl.ds(c * ROWS_B + s * BT + qq * IC, IC)], idxs)
      pltpu.sync_copy(dstm_hbm.at[pl.ds(s * BT + qq * IC, IC)], idxd)

      def body(jj, carry2):
        j0 = 2 * jj
        d0 = pltpu.async_copy(hp_hbm.at[idxs.at[j0]], gb0, sem0)
        d1 = pltpu.async_copy(hp_hbm.at[idxs.at[j0 + 1]], gb1, sem1)
        d0.wait()
        pltpu.sync_copy(gb0, acc_sp.at[idxd.at[j0]], add=True)
        d1.wait()
        pltpu.sync_copy(gb1, acc_sp.at[idxd.at[j0 + 1]], add=True)
        return carry2
      return lax.fori_loop(0, IC // 2, body, carry)
    lax.fori_loop(0, BT // IC, chunk, 0)
    plsc.subcore_barrier()
    pltpu.sync_copy(acc_sp.at[pl.ds(s * SLAB, SLAB)],
                    s_out.at[pl.ds(coff + s * SLAB, SLAB)])
    plsc.subcore_barrier()


def _dis_block(dp_ref):
  deg = dp_ref[0, :, 0:1] + dp_ref[1, :, 0:1]
  return lax.rsqrt(1.0 + deg)


def _m1_body(x_ref, w1_ref, dp0_ref, dp1_ref, o0_ref, o1_ref):
  xb = jnp.maximum(x_ref[...], 0.0)
  h = jnp.dot(xb, w1_ref[...], preferred_element_type=jnp.float32)
  for dp_ref, o_ref in ((dp0_ref, o0_ref), (dp1_ref, o1_ref)):
    dis = _dis_block(dp_ref)
    o_ref[0, :, :] = h[:, :AA] * dis
    o_ref[1, :, :] = h[:, AA:] * dis


def _m1(x0, w1, dp0, dp1):
  return pl.pallas_call(
      _m1_body,
      grid=(GRID,),
      in_specs=[
          pl.BlockSpec((BN, DD), lambda i: (i, 0)),
          pl.BlockSpec((DD, DD), lambda i: (0, 0)),
          pl.BlockSpec((2, BN, 16), lambda i: (0, i, 0)),
          pl.BlockSpec((2, BN, 16), lambda i: (0, i, 0)),
      ],
      out_specs=[pl.BlockSpec((2, BN, AA), lambda i: (0, i, 0))] * 2,
      out_shape=[jax.ShapeDtypeStruct((2, NPAD, AA), jnp.float32)] * 2,
  )(x0, w1, dp0, dp1)


def _m2_body(s1_ref, w2_ref, b1_ref, dp_ref, o_ref):
  dis = _dis_block(dp_ref)
  a1 = jnp.concatenate([s1_ref[0], s1_ref[1]], axis=1) * dis + b1_ref[...]
  h = jnp.dot(jnp.maximum(a1, 0.0), w2_ref[...],
              preferred_element_type=jnp.float32)
  o_ref[0, :, :] = h[:, :AA] * dis
  o_ref[1, :, :] = h[:, AA:] * dis


def _m2(s1, w2, b1, dp):
  return pl.pallas_call(
      _m2_body,
      grid=(GRID,),
      in_specs=[
          pl.BlockSpec((2, BN, AA), lambda i: (0, i, 0)),
          pl.BlockSpec((DD, DD), lambda i: (0, 0)),
          pl.BlockSpec((1, DD), lambda i: (0, 0)),
          pl.BlockSpec((2, BN, 16), lambda i: (0, i, 0)),
      ],
      out_specs=pl.BlockSpec((2, BN, AA), lambda i: (0, i, 0)),
      out_shape=jax.ShapeDtypeStruct((2, NPAD, AA), jnp.float32),
  )(s1, w2, b1, dp)


def _score_body(s20_ref, s21_ref, dp0_ref, dp1_ref, b2_ref,
                wsem_ref, bsem_ref, qsem_ref, att_ref, acc_ref):
  i = pl.program_id(0)

  @pl.when(i == 0)
  def _():
    acc_ref[0, 0] = 0.0
    acc_ref[0, 1] = 0.0

  for g, (s2_ref, dp_ref) in enumerate(((s20_ref, dp0_ref),
                                        (s21_ref, dp1_ref))):
    dis = _dis_block(dp_ref)
    out = jnp.maximum(
        jnp.concatenate([s2_ref[0], s2_ref[1]], axis=1) * dis + b2_ref[...],
        0.0)
    hp = jnp.tanh(jnp.dot(out, wsem_ref[...],
                          preferred_element_type=jnp.float32) + bsem_ref[...])
    acc_ref[0, g] += jnp.sum(hp * qsem_ref[...])

  @pl.when(i == GRID - 1)
  def _():
    s0 = acc_ref[0, 0] / NN
    s1 = acc_ref[0, 1] / NN
    m = jnp.maximum(s0, s1)
    e0 = jnp.exp(s0 - m)
    e1 = jnp.exp(s1 - m)
    att_ref[0, 0] = e0 / (e0 + e1)
    att_ref[0, 1] = e1 / (e0 + e1)


def _score(s20, s21, dp0, dp1, b2, wsem, bsem, qsem):
  return pl.pallas_call(
      _score_body,
      grid=(GRID,),
      in_specs=[
          pl.BlockSpec((2, BN, AA), lambda i: (0, i, 0)),
          pl.BlockSpec((2, BN, AA), lambda i: (0, i, 0)),
          pl.BlockSpec((2, BN, 16), lambda i: (0, i, 0)),
          pl.BlockSpec((2, BN, 16), lambda i: (0, i, 0)),
          pl.BlockSpec((1, DD), lambda i: (0, 0)),
          pl.BlockSpec((DD, AA), lambda i: (0, 0)),
          pl.BlockSpec((1, AA), lambda i: (0, 0)),
          pl.BlockSpec((1, AA), lambda i: (0, 0)),
      ],
      out_specs=pl.BlockSpec(memory_space=pltpu.SMEM),
      out_shape=jax.ShapeDtypeStruct((1, 2), jnp.float32),
      scratch_shapes=[pltpu.SMEM((1, 2), jnp.float32)],
  )(s20, s21, dp0, dp1, b2, wsem, bsem, qsem)


def _final_body(att_ref, s20_ref, s21_ref, dp0_ref, dp1_ref, b2_ref, o_ref):
  outs = []
  for s2_ref, dp_ref in ((s20_ref, dp0_ref), (s21_ref, dp1_ref)):
    dis = _dis_block(dp_ref)
    outs.append(jnp.maximum(
        jnp.concatenate([s2_ref[0], s2_ref[1]], axis=1) * dis + b2_ref[...],
        0.0))
  o_ref[...] = outs[0] * att_ref[0, 0] + outs[1] * att_ref[0, 1]


def _final(att, s20, s21, dp0, dp1, b2):
  return pl.pallas_call(
      _final_body,
      grid=(GRID,),
      in_specs=[
          pl.BlockSpec(memory_space=pltpu.SMEM),
          pl.BlockSpec((2, BN, AA), lambda i: (0, i, 0)),
          pl.BlockSpec((2, BN, AA), lambda i: (0, i, 0)),
          pl.BlockSpec((2, BN, 16), lambda i: (0, i, 0)),
          pl.BlockSpec((2, BN, 16), lambda i: (0, i, 0)),
          pl.BlockSpec((1, DD), lambda i: (0, 0)),
      ],
      out_specs=pl.BlockSpec((BN, DD), lambda i: (i, 0)),
      out_shape=jax.ShapeDtypeStruct((NN, DD), jnp.float32),
  )(att, s20, s21, dp0, dp1, b2)


def _edge_arrays(ei):
  src = ei[0].astype(jnp.int32)
  dst = ei[1].astype(jnp.int32)
  pad_s = jnp.zeros((EPAD - EE,), jnp.int32)
  pad_d = jnp.full((EPAD - EE,), NN, jnp.int32)
  src_p = jnp.concatenate([src, pad_s]).reshape(ROWS_B, KB)
  dst_p = jnp.concatenate([dst, pad_d]).reshape(ROWS_B, KB)
  srcm = jnp.concatenate([src_p, src_p + NPAD], axis=0)
  return srcm, dst_p


def kernel(x, edge_index_0, edge_index_1, emb_table, W1, b1, W2, b2,
           Wsem, bsem, qsem):
  x_i = x.astype(jnp.int32)
  x2d = jnp.concatenate(
      [x_i, jnp.zeros((NPAD - NN,), jnp.int32)]).reshape(XROWS, KX)
  src0m, dst0m = _edge_arrays(edge_index_0)
  src1m, dst1m = _edge_arrays(edge_index_1)

  x0, hist0, hist1 = _sc_prep(emb_table, x2d, dst0m, dst1m)
  dp0 = hist0.reshape(2, NPAD, 16)
  dp1 = hist1.reshape(2, NPAD, 16)

  h1p0, h1p1 = _m1(x0, W1, dp0, dp1)

  s10f, s11f = _sc_scatter2(h1p0.reshape(2 * NPAD, AA),
                            h1p1.reshape(2 * NPAD, AA),
                            src0m, dst0m, src1m, dst1m)
  s10 = s10f.reshape(2, NPAD, AA)
  s11 = s11f.reshape(2, NPAD, AA)

  b1r = b1.reshape(1, DD)
  h2p0 = _m2(s10, W2, b1r, dp0)
  h2p1 = _m2(s11, W2, b1r, dp1)

  s20f, s21f = _sc_scatter2(h2p0.reshape(2 * NPAD, AA),
                            h2p1.reshape(2 * NPAD, AA),
                            src0m, dst0m, src1m, dst1m)
  s20 = s20f.reshape(2, NPAD, AA)
  s21 = s21f.reshape(2, NPAD, AA)

  b2r = b2.reshape(1, DD)
  att = _score(s20, s21, dp0, dp1, b2r, Wsem, bsem, qsem)
  h_emb = _final(att, s20, s21, dp0, dp1, b2r)
  return (h_emb, att)
